# Initial kernel scaffold; baseline (speedup 1.0000x reference)
import jax
import jax.numpy as jnp
from jax.experimental import pallas as pl


def kernel(x, edge_index, batch_ids, atom_emb, Ws, bs, gammas, betas, W1, b1, W2, b2, W3, b3):
    raise NotImplementedError("write your pallas kernel here")


# speedup vs baseline: 6.7414x; 6.7414x over previous
import functools

import jax
import jax.numpy as jnp
from jax import lax
from jax.experimental import pallas as pl
from jax.experimental.pallas import tpu as pltpu
from jax.experimental.pallas import tpu_sc as plsc

N = 10000
E = 320000
H = 128
L = 4
G = 64
VOCAB = 128
NFEAT = 9

NC = 2
NS = 16
NW = NC * NS

NP = 12288
NODES_W = NP // NW
EMB_CHUNKS = NODES_W // 128

EP = 323584
EDGE_CHUNKS = EP // NW // 128
PAD_DST = N + 8
NACC = 10112
ROWS_T = NACC // NS

_MESH = plsc.VectorSubcoreMesh(core_axis_name="c", subcore_axis_name="s")
_SC_PARAMS = pltpu.CompilerParams(use_tc_tiling_on_sc=False)


def _fill_rows(ref, nrows, ncols, value):
    v = jnp.full((16,), value, dtype=jnp.float32)

    def body(i, _):
        for t in range(ncols // 16):
            ref[i, pl.ds(t * 16, 16)] = v
        return 0

    lax.fori_loop(0, nrows, body, 0)


@functools.partial(
    pl.kernel,
    out_type=(
        jax.ShapeDtypeStruct((NP, H), jnp.float32),
        jax.ShapeDtypeStruct((NC, NACC, 16), jnp.float32),
    ),
    mesh=_MESH,
    compiler_params=_SC_PARAMS,
    scratch_types=dict(
        hacc=pltpu.VMEM_SHARED((NP // NC, H), jnp.float32),
        dacc=pltpu.VMEM_SHARED((NACC, 16), jnp.float32),
        idxrow=pltpu.VMEM((1, 128), jnp.int32),
        iotab=pltpu.VMEM((EMB_CHUNKS, 128), jnp.int32),
        gtmp=pltpu.VMEM((128, H), jnp.float32),
        dstb=pltpu.VMEM((EDGE_CHUNKS, 128), jnp.int32),
        onesb=pltpu.VMEM((128, 16), jnp.float32),
        zbuf=pltpu.VMEM((ROWS_T, 16), jnp.float32),
    ),
)
def _sc_encode(emb_flat, xt_pad, dst3, h_out, deg_out,
               hacc, dacc, idxrow, iotab, gtmp, dstb, onesb, zbuf):
    c = lax.axis_index("c")
    s = lax.axis_index("s")
    w = c * NS + s

    _fill_rows(onesb, 128, 16, 1.0)
    _fill_rows(zbuf, ROWS_T, 16, 0.0)
    base_iota = lax.iota(jnp.int32, 16)
    for k in range(EMB_CHUNKS):
        for t in range(8):
            iotab[k, pl.ds(t * 16, 16)] = base_iota + (s * NODES_W + k * 128 + t * 16)

    pltpu.sync_copy(zbuf, dacc.at[pl.ds(s * ROWS_T, ROWS_T)])
    plsc.subcore_barrier()

    for f in range(NFEAT):
        for k in range(EMB_CHUNKS):
            row = f * (NP // 128) + w * EMB_CHUNKS + k
            pltpu.sync_copy(xt_pad.at[row], idxrow)
            for t in range(8):
                idxrow[0, pl.ds(t * 16, 16)] = (
                    idxrow[0, pl.ds(t * 16, 16)] + f * VOCAB)
            pltpu.sync_copy(emb_flat.at[idxrow.at[0]], gtmp)
            pltpu.sync_copy(gtmp, hacc.at[iotab.at[k]], add=(f > 0))

    pltpu.sync_copy(dst3.at[w], dstb)

    def deg_body(j, _):
        pltpu.sync_copy(onesb, dacc.at[dstb.at[j]], add=True)
        return 0

    lax.fori_loop(0, EDGE_CHUNKS, deg_body, 0)
    plsc.subcore_barrier()

    pltpu.sync_copy(hacc.at[pl.ds(s * NODES_W, NODES_W)],
                    h_out.at[pl.ds((c * NS + s) * NODES_W, NODES_W)])
    pltpu.sync_copy(dacc.at[pl.ds(s * ROWS_T, ROWS_T)],
                    deg_out.at[c, pl.ds(s * ROWS_T, ROWS_T)])


@functools.partial(
    pl.kernel,
    out_type=jax.ShapeDtypeStruct((NC, NACC, H), jnp.float32),
    mesh=_MESH,
    compiler_params=_SC_PARAMS,
    scratch_types=dict(
        acc=pltpu.VMEM_SHARED((NACC, H), jnp.float32),
        srcb=pltpu.VMEM((EDGE_CHUNKS, 128), jnp.int32),
        dstb=pltpu.VMEM((EDGE_CHUNKS, 128), jnp.int32),
        gtmp=pltpu.VMEM((128, H), jnp.float32),
    ),
)
def _sc_agg(hs, src3, dst3, acc_out, acc, srcb, dstb, gtmp):
    c = lax.axis_index("c")
    s = lax.axis_index("s")
    w = c * NS + s

    _fill_rows(gtmp, 128, H, 0.0)
    for r0 in range(0, ROWS_T, 128):
        rows = min(128, ROWS_T - r0)
        pltpu.sync_copy(gtmp.at[pl.ds(0, rows)],
                        acc.at[pl.ds(s * ROWS_T + r0, rows)])
    plsc.subcore_barrier()

    pltpu.sync_copy(src3.at[w], srcb)
    pltpu.sync_copy(dst3.at[w], dstb)

    def body(j, _):
        pltpu.sync_copy(hs.at[srcb.at[j]], gtmp)
        pltpu.sync_copy(gtmp, acc.at[dstb.at[j]], add=True)
        return 0

    lax.fori_loop(0, EDGE_CHUNKS, body, 0)
    plsc.subcore_barrier()

    pltpu.sync_copy(acc.at[pl.ds(s * ROWS_T, ROWS_T)],
                    acc_out.at[c, pl.ds(s * ROWS_T, ROWS_T)])


def _tc_prep_body(h_ref, deg2_ref, hs_ref, norm_ref, invdeg_ref):
    deg = deg2_ref[0] + deg2_ref[1] + 1.0
    norm = lax.rsqrt(deg)
    norm_ref[...] = norm
    invdeg_ref[...] = 1.0 / deg
    hs_ref[...] = h_ref[...] * norm


def _tc_layer_body(acc0_ref, acc1_ref, h_ref, norm_ref, invdeg_ref,
                   w_ref, b_ref, gamma_ref, beta_ref, hn_ref, hs_ref):
    h = h_ref[...]
    agg = (acc0_ref[...] + acc1_ref[...]) * norm_ref[...] + h * invdeg_ref[...]
    hp = jnp.dot(agg, w_ref[...], preferred_element_type=jnp.float32) + b_ref[...]
    mean = jnp.mean(hp, axis=0, keepdims=True)
    var = jnp.mean((hp - mean) * (hp - mean), axis=0, keepdims=True)
    hb = (hp - mean) * lax.rsqrt(var + 1e-5) * gamma_ref[...] + beta_ref[...]
    hn = jnp.maximum(hb, 0.0) + h
    hn_ref[...] = hn
    hs_ref[...] = hn * norm_ref[...]


def _tc_final_body(h_ref, bid_ref, w1_ref, b1_ref, w2_ref, b2_ref,
                   w3_ref, b3_ref, out_ref):
    bid = bid_ref[...]
    gids = lax.broadcasted_iota(jnp.int32, (N, G), 1)
    mask = (bid == gids).astype(jnp.float32)
    sums = lax.dot_general(mask, h_ref[...], (((0,), (0,)), ((), ())),
                           preferred_element_type=jnp.float32)
    counts = jnp.sum(mask, axis=0)[:, None]
    pooled = sums / jnp.maximum(counts, 1.0)
    z = jnp.maximum(jnp.dot(pooled, w1_ref[...],
                            preferred_element_type=jnp.float32) + b1_ref[...], 0.0)
    z = jnp.maximum(jnp.dot(z, w2_ref[...],
                            preferred_element_type=jnp.float32) + b2_ref[...], 0.0)
    out_ref[...] = jnp.dot(z, w3_ref[...],
                           preferred_element_type=jnp.float32) + b3_ref[...]


_tc_prep = pl.pallas_call(
    _tc_prep_body,
    out_shape=(
        jax.ShapeDtypeStruct((N, H), jnp.float32),
        jax.ShapeDtypeStruct((N, 1), jnp.float32),
        jax.ShapeDtypeStruct((N, 1), jnp.float32),
    ),
)

_tc_layer = pl.pallas_call(
    _tc_layer_body,
    out_shape=(
        jax.ShapeDtypeStruct((N, H), jnp.float32),
        jax.ShapeDtypeStruct((N, H), jnp.float32),
    ),
)

_tc_final = pl.pallas_call(
    _tc_final_body,
    out_shape=jax.ShapeDtypeStruct((G, 1), jnp.float32),
)


def kernel(x, edge_index, batch_ids, atom_emb, Ws, bs, gammas, betas,
           W1, b1, W2, b2, W3, b3):
    emb_flat = atom_emb.reshape(NFEAT * VOCAB, H)
    xt_pad = jnp.pad(x.astype(jnp.int32).T,
                     ((0, 0), (0, NP - N))).reshape(NFEAT * NP // 128, 1, 128)
    src = edge_index[0].astype(jnp.int32)
    dst = edge_index[1].astype(jnp.int32)
    src3 = jnp.pad(src, (0, EP - E)).reshape(NW, EDGE_CHUNKS, 128)
    dst3 = jnp.pad(dst, (0, EP - E),
                   constant_values=PAD_DST).reshape(NW, EDGE_CHUNKS, 128)

    h_pad, deg2 = _sc_encode(emb_flat, xt_pad, dst3)
    h = h_pad[:N]
    deg2s = deg2[:, :N, :1]

    hs, norm, invdeg = _tc_prep(h, deg2s)

    for i in range(L):
        acc2 = _sc_agg(hs, src3, dst3)
        h, hs = _tc_layer(acc2[0, :N], acc2[1, :N], h, norm, invdeg,
                          Ws[i], bs[i][None, :], gammas[i][None, :],
                          betas[i][None, :])

    out = _tc_final(h, batch_ids.astype(jnp.int32)[:, None],
                    W1, b1[None, :], W2, b2[None, :], W3, b3[None, :])
    return out
